# baseline (speedup 1.0000x reference)
import functools

import jax
import jax.numpy as jnp
from jax import lax
from jax.experimental import pallas as pl
from jax.experimental.pallas import tpu as pltpu
from jax.experimental.pallas import tpu_sc as plsc

_S, _B, _D = 2048, 4, 1024
_NC, _NS = 2, 16
_NW = _NC * _NS
_S_PER_W = _S // _NW
_CH = 8
_N_CH = _S_PER_W // _CH
_NB = 3
_L = 16


def _sc_body(x_hbm, w_hbm, out_hbm, x_v0, x_v1, x_v2, w_v0, w_v1, w_v2,
             six0, six1, six2, siw0, siw1, siw2, so0, so1, so2):
    xs, ws = (x_v0, x_v1, x_v2), (w_v0, w_v1, w_v2)
    six, siw, so = (six0, six1, six2), (siw0, siw1, siw2), (so0, so1, so2)

    cid = lax.axis_index("c")
    sid = lax.axis_index("s")
    wid = sid * _NC + cid
    s_base = wid * _S_PER_W

    def in_copies(c, b):
        s0 = s_base + c * _CH
        return (
            pltpu.make_async_copy(x_hbm.at[pl.ds(s0, _CH)], xs[b], six[b]),
            pltpu.make_async_copy(w_hbm.at[pl.ds(s0, _CH)], ws[b], siw[b]),
        )

    def out_copy(c, b):
        s0 = s_base + c * _CH
        return pltpu.make_async_copy(xs[b], out_hbm.at[pl.ds(s0, _CH)], so[b])

    def start_in(c, b):
        cx, cw = in_copies(c, b)
        cx.start()
        cw.start()

    start_in(0, 0)
    start_in(1, 1)

    for c in range(_N_CH):
        b = c % _NB
        cx, cw = in_copies(c, b)
        cx.wait()
        cw.wait()

        def j_body(j, carry, b=b):
            dj = pl.ds(j * _L, _L)
            for s in range(_CH):
                wv = ws[b][s, dj]
                for bb in range(_B):
                    xs[b][s, bb, dj] += wv
            return carry

        lax.fori_loop(0, _D // _L, j_body, 0)

        out_copy(c, b).start()

        if c + 2 < _N_CH:
            if c >= 1:
                out_copy(c - 1, (c - 1) % _NB).wait()
            start_in(c + 2, (c + 2) % _NB)

    out_copy(_N_CH - 3, (_N_CH - 3) % _NB).wait()
    out_copy(_N_CH - 2, (_N_CH - 2) % _NB).wait()
    out_copy(_N_CH - 1, (_N_CH - 1) % _NB).wait()


def kernel(x, pos_embed_weight):
    mesh = plsc.VectorSubcoreMesh(core_axis_name="c", subcore_axis_name="s")
    run = functools.partial(
        pl.kernel,
        mesh=mesh,
        out_type=jax.ShapeDtypeStruct((_S, _B, _D), jnp.float32),
        scratch_types=[
            pltpu.VMEM((_CH, _B, _D), jnp.float32),
            pltpu.VMEM((_CH, _B, _D), jnp.float32),
            pltpu.VMEM((_CH, _B, _D), jnp.float32),
            pltpu.VMEM((_CH, _D), jnp.float32),
            pltpu.VMEM((_CH, _D), jnp.float32),
            pltpu.VMEM((_CH, _D), jnp.float32),
            pltpu.SemaphoreType.DMA,
            pltpu.SemaphoreType.DMA,
            pltpu.SemaphoreType.DMA,
            pltpu.SemaphoreType.DMA,
            pltpu.SemaphoreType.DMA,
            pltpu.SemaphoreType.DMA,
            pltpu.SemaphoreType.DMA,
            pltpu.SemaphoreType.DMA,
            pltpu.SemaphoreType.DMA,
        ],
    )(_sc_body)
    return run(x, pos_embed_weight)


# speedup vs baseline: 2.1631x; 2.1631x over previous
import jax
import jax.numpy as jnp
from jax.experimental import pallas as pl


_S_BLK = 512


def _pe_add_kernel(x_ref, w_ref, o_ref):
    w = w_ref[...]
    for b in range(x_ref.shape[1]):
        o_ref[:, b, :] = x_ref[:, b, :] + w


def kernel(x, pos_embed_weight):
    seq_len, batch, d_model = x.shape
    grid = (seq_len // _S_BLK,)
    return pl.pallas_call(
        _pe_add_kernel,
        grid=grid,
        in_specs=[
            pl.BlockSpec((_S_BLK, batch, d_model), lambda i: (i, 0, 0)),
            pl.BlockSpec((_S_BLK, d_model), lambda i: (i, 0)),
        ],
        out_specs=pl.BlockSpec((_S_BLK, batch, d_model), lambda i: (i, 0, 0)),
        out_shape=jax.ShapeDtypeStruct(x.shape, x.dtype),
    )(x, pos_embed_weight)
